# Initial kernel scaffold; baseline (speedup 1.0000x reference)
import jax
import jax.numpy as jnp
from jax.experimental import pallas as pl


def kernel(f_atoms, f_bonds, a2b, b2a, b2revb, undirected_b2a, directed_b2a, parity_atoms, mask, W_i, W_ih, W_hh, b_ih, b_hh, W_o, b_o):
    raise NotImplementedError("write your pallas kernel here")


# speedup vs baseline: 1.7291x; 1.7291x over previous
import functools

import jax
import jax.numpy as jnp
from jax import lax
from jax.experimental import pallas as pl
from jax.experimental.pallas import tpu as pltpu
from jax.experimental.pallas import tpu_sc as plsc

N_ATOMS = 10000
N_BONDS = 320000
MAX_NB = 32
ATOM_FDIM = 128
BOND_FDIM = 16
HIDDEN = 128
DEPTH = 4

NC, NS, NL = 2, 16, 16
NW = NC * NS
NVH = HIDDEN // NL

_SC_MESH = plsc.VectorSubcoreMesh(core_axis_name="c", subcore_axis_name="s")

CA = 4
N_CHUNKS_A = N_ATOMS // CA


def _sc_gather_sum_body(msg_hbm, a2b_hbm, out_hbm, idx_v, rows_v, out_v, sem):
    wid = lax.axis_index("s") * NC + lax.axis_index("c")
    n_base = N_CHUNKS_A // NW
    n_extra = N_CHUNKS_A - n_base * NW
    n_i = n_base + jnp.where(wid < n_extra, 1, 0)

    def chunk_body(i, carry):
        c = wid + i * NW
        pltpu.sync_copy(a2b_hbm.at[pl.ds(c * (CA * MAX_NB), CA * MAX_NB)], idx_v)
        pltpu.async_copy(msg_hbm.at[idx_v], rows_v, sem).wait()
        for a in range(CA):
            accs = tuple(jnp.zeros((NL,), jnp.float32) for _ in range(NVH))

            def red(k, acc, a=a):
                return tuple(acc[j] + rows_v[a * MAX_NB + k, pl.ds(j * NL, NL)]
                             for j in range(NVH))

            accs = lax.fori_loop(0, MAX_NB, red, accs)
            for j in range(NVH):
                out_v[a, pl.ds(j * NL, NL)] = accs[j]
        pltpu.sync_copy(out_v, out_hbm.at[pl.ds(c * CA, CA)])
        return carry

    lax.fori_loop(0, n_i, chunk_body, 0)


_sc_gather_sum = functools.partial(
    pl.kernel,
    mesh=_SC_MESH,
    out_type=jax.ShapeDtypeStruct((N_ATOMS, HIDDEN), jnp.float32),
    scratch_types=[
        pltpu.VMEM((CA * MAX_NB,), jnp.int32),
        pltpu.VMEM((CA * MAX_NB, HIDDEN), jnp.float32),
        pltpu.VMEM((CA, HIDDEN), jnp.float32),
        pltpu.SemaphoreType.DMA,
    ],
)(_sc_gather_sum_body)


CB = 128
N_CHUNKS_B = N_BONDS // CB


def _sc_gather_diff_body(amsg_hbm, msg_hbm, b2a_hbm, b2revb_hbm, out_hbm,
                         idxa_v, idxr_v, rowsa_v, rowsr_v, sema, semr):
    wid = lax.axis_index("s") * NC + lax.axis_index("c")
    n_base = N_CHUNKS_B // NW
    n_extra = N_CHUNKS_B - n_base * NW
    n_i = n_base + jnp.where(wid < n_extra, 1, 0)

    def chunk_body(i, carry):
        c = wid + i * NW
        pltpu.sync_copy(b2a_hbm.at[pl.ds(c * CB, CB)], idxa_v)
        pltpu.sync_copy(b2revb_hbm.at[pl.ds(c * CB, CB)], idxr_v)
        cpa = pltpu.async_copy(amsg_hbm.at[idxa_v], rowsa_v, sema)
        cpr = pltpu.async_copy(msg_hbm.at[idxr_v], rowsr_v, semr)
        cpa.wait()
        cpr.wait()

        def sub(b, carry2):
            for j in range(NVH):
                rowsa_v[b, pl.ds(j * NL, NL)] = (
                    rowsa_v[b, pl.ds(j * NL, NL)] - rowsr_v[b, pl.ds(j * NL, NL)])
            return carry2

        lax.fori_loop(0, CB, sub, 0)
        pltpu.sync_copy(rowsa_v, out_hbm.at[pl.ds(c * CB, CB)])
        return carry

    lax.fori_loop(0, n_i, chunk_body, 0)


_sc_gather_diff = functools.partial(
    pl.kernel,
    mesh=_SC_MESH,
    out_type=jax.ShapeDtypeStruct((N_BONDS, HIDDEN), jnp.float32),
    scratch_types=[
        pltpu.VMEM((CB,), jnp.int32),
        pltpu.VMEM((CB,), jnp.int32),
        pltpu.VMEM((CB, HIDDEN), jnp.float32),
        pltpu.VMEM((CB, HIDDEN), jnp.float32),
        pltpu.SemaphoreType.DMA,
        pltpu.SemaphoreType.DMA,
    ],
)(_sc_gather_diff_body)


BN = 2000
BA = 2000


def _tc_pre_body(fb_ref, wit_ref, wiht_ref, bih_ref, inp_ref, gi_ref):
    inp = jnp.dot(fb_ref[...], wit_ref[...], preferred_element_type=jnp.float32)
    inp_ref[...] = inp
    gi_ref[...] = (
        jnp.dot(inp, wiht_ref[...], preferred_element_type=jnp.float32)
        + bih_ref[...])


def _tc_pre(f_bonds, wit, wiht, bih):
    return pl.pallas_call(
        _tc_pre_body,
        grid=(N_BONDS // BN,),
        in_specs=[
            pl.BlockSpec((BN, BOND_FDIM), lambda i: (i, 0)),
            pl.BlockSpec((BOND_FDIM, HIDDEN), lambda i: (0, 0)),
            pl.BlockSpec((HIDDEN, 3 * HIDDEN), lambda i: (0, 0)),
            pl.BlockSpec((1, 3 * HIDDEN), lambda i: (0, 0)),
        ],
        out_specs=[
            pl.BlockSpec((BN, HIDDEN), lambda i: (i, 0)),
            pl.BlockSpec((BN, 3 * HIDDEN), lambda i: (i, 0)),
        ],
        out_shape=[
            jax.ShapeDtypeStruct((N_BONDS, HIDDEN), jnp.float32),
            jax.ShapeDtypeStruct((N_BONDS, 3 * HIDDEN), jnp.float32),
        ],
    )(f_bonds, wit, wiht, bih)


def _tc_gru_body(m_ref, gi_ref, whht_ref, bhh_ref, out_ref):
    m = m_ref[...]
    gh = (jnp.dot(m, whht_ref[...], preferred_element_type=jnp.float32)
          + bhh_ref[...])
    gi = gi_ref[...]
    r = jax.nn.sigmoid(gi[:, :HIDDEN] + gh[:, :HIDDEN])
    z = jax.nn.sigmoid(gi[:, HIDDEN:2 * HIDDEN] + gh[:, HIDDEN:2 * HIDDEN])
    n = jnp.tanh(gi[:, 2 * HIDDEN:] + r * gh[:, 2 * HIDDEN:])
    out_ref[...] = (1.0 - z) * n + z * m

    @pl.when(pl.program_id(0) == 0)
    def _zero_row0():
        out_ref[0:1, :] = jnp.zeros((1, HIDDEN), jnp.float32)


def _tc_gru(m_in, gi, whht, bhh):
    return pl.pallas_call(
        _tc_gru_body,
        grid=(N_BONDS // BN,),
        in_specs=[
            pl.BlockSpec((BN, HIDDEN), lambda i: (i, 0)),
            pl.BlockSpec((BN, 3 * HIDDEN), lambda i: (i, 0)),
            pl.BlockSpec((HIDDEN, 3 * HIDDEN), lambda i: (0, 0)),
            pl.BlockSpec((1, 3 * HIDDEN), lambda i: (0, 0)),
        ],
        out_specs=pl.BlockSpec((BN, HIDDEN), lambda i: (i, 0)),
        out_shape=jax.ShapeDtypeStruct((N_BONDS, HIDDEN), jnp.float32),
    )(m_in, gi, whht, bhh)


def _tc_out_body(fa_ref, am_ref, woa_ref, wom_ref, bo_ref, mask_ref, o_ref):
    h = (jnp.dot(fa_ref[...], woa_ref[...], preferred_element_type=jnp.float32)
         + jnp.dot(am_ref[...], wom_ref[...], preferred_element_type=jnp.float32)
         + bo_ref[...])
    o_ref[...] = jnp.maximum(h, 0.0) * mask_ref[...]


def _tc_out(f_atoms, amsg, woat, womt, bo, mask):
    return pl.pallas_call(
        _tc_out_body,
        grid=(N_ATOMS // BA,),
        in_specs=[
            pl.BlockSpec((BA, ATOM_FDIM), lambda i: (i, 0)),
            pl.BlockSpec((BA, HIDDEN), lambda i: (i, 0)),
            pl.BlockSpec((ATOM_FDIM, HIDDEN), lambda i: (0, 0)),
            pl.BlockSpec((HIDDEN, HIDDEN), lambda i: (0, 0)),
            pl.BlockSpec((1, HIDDEN), lambda i: (0, 0)),
            pl.BlockSpec((BA, 1), lambda i: (i, 0)),
        ],
        out_specs=pl.BlockSpec((BA, HIDDEN), lambda i: (i, 0)),
        out_shape=jax.ShapeDtypeStruct((N_ATOMS, HIDDEN), jnp.float32),
    )(f_atoms, amsg, woat, womt, bo, mask)


def kernel(f_atoms, f_bonds, a2b, b2a, b2revb, undirected_b2a, directed_b2a,
           parity_atoms, mask, W_i, W_ih, W_hh, b_ih, b_hh, W_o, b_o):
    wit = W_i.T
    wiht = W_ih.T
    whht = W_hh.T
    woat = W_o[:, :ATOM_FDIM].T
    womt = W_o[:, ATOM_FDIM:].T
    bih = b_ih.reshape(1, 3 * HIDDEN)
    bhh = b_hh.reshape(1, 3 * HIDDEN)
    bo = b_o.reshape(1, HIDDEN)
    a2b_flat = a2b.reshape(-1)

    inp, gi = _tc_pre(f_bonds, wit, wiht, bih)
    message = inp
    for _ in range(DEPTH - 1):
        amsg = _sc_gather_sum(message, a2b_flat)
        m_in = _sc_gather_diff(amsg, message, b2a, b2revb)
        message = _tc_gru(m_in, gi, whht, bhh)
    amsg = _sc_gather_sum(message, a2b_flat)
    return _tc_out(f_atoms, amsg, woat, womt, bo, mask)


# speedup vs baseline: 1.8449x; 1.0669x over previous
import functools

import jax
import jax.numpy as jnp
from jax import lax
from jax.experimental import pallas as pl
from jax.experimental.pallas import tpu as pltpu
from jax.experimental.pallas import tpu_sc as plsc

N_ATOMS = 10000
N_BONDS = 320000
MAX_NB = 32
ATOM_FDIM = 128
BOND_FDIM = 16
HIDDEN = 128
DEPTH = 4

NC, NS, NL = 2, 16, 16
NW = NC * NS
NVH = HIDDEN // NL

_SC_MESH = plsc.VectorSubcoreMesh(core_axis_name="c", subcore_axis_name="s")

CA = 4
N_CHUNKS_A = N_ATOMS // CA


def _sc_gather_sum_body(msg_hbm, a2b_hbm, out_hbm, idx_v, rows_v, out_v, sem):
    wid = lax.axis_index("s") * NC + lax.axis_index("c")
    n_base = N_CHUNKS_A // NW
    n_extra = N_CHUNKS_A - n_base * NW
    n_i = n_base + jnp.where(wid < n_extra, 1, 0)

    def chunk_body(i, carry):
        c = wid + i * NW
        pltpu.sync_copy(a2b_hbm.at[pl.ds(c * (CA * MAX_NB), CA * MAX_NB)], idx_v)
        pltpu.async_copy(msg_hbm.at[idx_v], rows_v, sem).wait()
        for a in range(CA):
            accs = tuple(jnp.zeros((NL,), jnp.float32) for _ in range(NVH))

            def red(k, acc, a=a):
                return tuple(acc[j] + rows_v[a * MAX_NB + k, pl.ds(j * NL, NL)]
                             for j in range(NVH))

            accs = lax.fori_loop(0, MAX_NB, red, accs)
            for j in range(NVH):
                out_v[a, pl.ds(j * NL, NL)] = accs[j]
        pltpu.sync_copy(out_v, out_hbm.at[pl.ds(c * CA, CA)])
        return carry

    lax.fori_loop(0, n_i, chunk_body, 0)


_sc_gather_sum = functools.partial(
    pl.kernel,
    mesh=_SC_MESH,
    out_type=jax.ShapeDtypeStruct((N_ATOMS, HIDDEN), jnp.float32),
    scratch_types=[
        pltpu.VMEM((CA * MAX_NB,), jnp.int32),
        pltpu.VMEM((CA * MAX_NB, HIDDEN), jnp.float32),
        pltpu.VMEM((CA, HIDDEN), jnp.float32),
        pltpu.SemaphoreType.DMA,
    ],
)(_sc_gather_sum_body)


CB = 128
N_CHUNKS_B = N_BONDS // CB


def _sc_gather_diff_body(amsg_hbm, msg_hbm, b2a_hbm, b2revb_hbm, out_hbm,
                         idxa_v, idxr_v, rowsa_v, rowsr_v, sema, semr):
    wid = lax.axis_index("s") * NC + lax.axis_index("c")
    n_base = N_CHUNKS_B // NW
    n_extra = N_CHUNKS_B - n_base * NW
    n_i = n_base + jnp.where(wid < n_extra, 1, 0)

    def chunk_body(i, carry):
        c = wid + i * NW
        pltpu.sync_copy(b2a_hbm.at[pl.ds(c * CB, CB)], idxa_v)
        pltpu.sync_copy(b2revb_hbm.at[pl.ds(c * CB, CB)], idxr_v)
        cpa = pltpu.async_copy(amsg_hbm.at[idxa_v], rowsa_v, sema)
        cpr = pltpu.async_copy(msg_hbm.at[idxr_v], rowsr_v, semr)
        cpa.wait()
        cpr.wait()

        def sub(b, carry2):
            for j in range(NVH):
                rowsa_v[b, pl.ds(j * NL, NL)] = (
                    rowsa_v[b, pl.ds(j * NL, NL)] - rowsr_v[b, pl.ds(j * NL, NL)])
            return carry2

        lax.fori_loop(0, CB, sub, 0)
        pltpu.sync_copy(rowsa_v, out_hbm.at[pl.ds(c * CB, CB)])
        return carry

    lax.fori_loop(0, n_i, chunk_body, 0)


_sc_gather_diff = functools.partial(
    pl.kernel,
    mesh=_SC_MESH,
    out_type=jax.ShapeDtypeStruct((N_BONDS, HIDDEN), jnp.float32),
    scratch_types=[
        pltpu.VMEM((CB,), jnp.int32),
        pltpu.VMEM((CB,), jnp.int32),
        pltpu.VMEM((CB, HIDDEN), jnp.float32),
        pltpu.VMEM((CB, HIDDEN), jnp.float32),
        pltpu.SemaphoreType.DMA,
        pltpu.SemaphoreType.DMA,
    ],
)(_sc_gather_diff_body)


BN = 2000
BA = 2000


def _tc_pre_body(fb_ref, wit_ref, inp_ref):
    inp_ref[...] = jnp.dot(fb_ref[...], wit_ref[...],
                           preferred_element_type=jnp.float32)


def _tc_pre(f_bonds, wit):
    return pl.pallas_call(
        _tc_pre_body,
        grid=(N_BONDS // BN,),
        in_specs=[
            pl.BlockSpec((BN, BOND_FDIM), lambda i: (i, 0)),
            pl.BlockSpec((BOND_FDIM, HIDDEN), lambda i: (0, 0)),
        ],
        out_specs=pl.BlockSpec((BN, HIDDEN), lambda i: (i, 0)),
        out_shape=jax.ShapeDtypeStruct((N_BONDS, HIDDEN), jnp.float32),
    )(f_bonds, wit)


def _tc_gru_body(m_ref, inp_ref, wiht_ref, bih_ref, whht_ref, bhh_ref, out_ref):
    m = m_ref[...]
    gh = (jnp.dot(m, whht_ref[...], preferred_element_type=jnp.float32)
          + bhh_ref[...])
    gi = (jnp.dot(inp_ref[...], wiht_ref[...],
                  preferred_element_type=jnp.float32)
          + bih_ref[...])
    r = jax.nn.sigmoid(gi[:, :HIDDEN] + gh[:, :HIDDEN])
    z = jax.nn.sigmoid(gi[:, HIDDEN:2 * HIDDEN] + gh[:, HIDDEN:2 * HIDDEN])
    n = jnp.tanh(gi[:, 2 * HIDDEN:] + r * gh[:, 2 * HIDDEN:])
    out_ref[...] = (1.0 - z) * n + z * m

    @pl.when(pl.program_id(0) == 0)
    def _zero_row0():
        out_ref[0:1, :] = jnp.zeros((1, HIDDEN), jnp.float32)


def _tc_gru(m_in, inp, wiht, bih, whht, bhh):
    return pl.pallas_call(
        _tc_gru_body,
        grid=(N_BONDS // BN,),
        in_specs=[
            pl.BlockSpec((BN, HIDDEN), lambda i: (i, 0)),
            pl.BlockSpec((BN, HIDDEN), lambda i: (i, 0)),
            pl.BlockSpec((HIDDEN, 3 * HIDDEN), lambda i: (0, 0)),
            pl.BlockSpec((1, 3 * HIDDEN), lambda i: (0, 0)),
            pl.BlockSpec((HIDDEN, 3 * HIDDEN), lambda i: (0, 0)),
            pl.BlockSpec((1, 3 * HIDDEN), lambda i: (0, 0)),
        ],
        out_specs=pl.BlockSpec((BN, HIDDEN), lambda i: (i, 0)),
        out_shape=jax.ShapeDtypeStruct((N_BONDS, HIDDEN), jnp.float32),
    )(m_in, inp, wiht, bih, whht, bhh)


def _tc_out_body(fa_ref, am_ref, woa_ref, wom_ref, bo_ref, mask_ref, o_ref):
    h = (jnp.dot(fa_ref[...], woa_ref[...], preferred_element_type=jnp.float32)
         + jnp.dot(am_ref[...], wom_ref[...], preferred_element_type=jnp.float32)
         + bo_ref[...])
    o_ref[...] = jnp.maximum(h, 0.0) * mask_ref[...]


def _tc_out(f_atoms, amsg, woat, womt, bo, mask):
    return pl.pallas_call(
        _tc_out_body,
        grid=(N_ATOMS // BA,),
        in_specs=[
            pl.BlockSpec((BA, ATOM_FDIM), lambda i: (i, 0)),
            pl.BlockSpec((BA, HIDDEN), lambda i: (i, 0)),
            pl.BlockSpec((ATOM_FDIM, HIDDEN), lambda i: (0, 0)),
            pl.BlockSpec((HIDDEN, HIDDEN), lambda i: (0, 0)),
            pl.BlockSpec((1, HIDDEN), lambda i: (0, 0)),
            pl.BlockSpec((BA, 1), lambda i: (i, 0)),
        ],
        out_specs=pl.BlockSpec((BA, HIDDEN), lambda i: (i, 0)),
        out_shape=jax.ShapeDtypeStruct((N_ATOMS, HIDDEN), jnp.float32),
    )(f_atoms, amsg, woat, womt, bo, mask)


def kernel(f_atoms, f_bonds, a2b, b2a, b2revb, undirected_b2a, directed_b2a,
           parity_atoms, mask, W_i, W_ih, W_hh, b_ih, b_hh, W_o, b_o):
    wit = W_i.T
    wiht = W_ih.T
    whht = W_hh.T
    woat = W_o[:, :ATOM_FDIM].T
    womt = W_o[:, ATOM_FDIM:].T
    bih = b_ih.reshape(1, 3 * HIDDEN)
    bhh = b_hh.reshape(1, 3 * HIDDEN)
    bo = b_o.reshape(1, HIDDEN)
    a2b_flat = a2b.reshape(-1)

    inp = _tc_pre(f_bonds, wit)
    message = inp
    for _ in range(DEPTH - 1):
        amsg = _sc_gather_sum(message, a2b_flat)
        m_in = _sc_gather_diff(amsg, message, b2a, b2revb)
        message = _tc_gru(m_in, inp, wiht, bih, whht, bhh)
    amsg = _sc_gather_sum(message, a2b_flat)
    return _tc_out(f_atoms, amsg, woat, womt, bo, mask)
